# baseline (speedup 1.0000x reference)
import jax
import jax.numpy as jnp
from jax.experimental import pallas as pl
from jax.experimental.pallas import tpu as pltpu

_H = 28
_W = 28
_P = _H * _W
_B_BLK = 256


def _axs_kernel(params_ref, x_ref, out_ref, a_ref):
    @pl.when(pl.program_id(0) == 0)
    def _build_a():
        pos0 = params_ref[0:1, :]
        pos1 = params_ref[1:2, :]
        sw = jnp.maximum(params_ref[2:3, :], 0.0)
        r0 = jnp.round(pos0)
        r1 = jnp.round(pos1)
        q = jax.lax.broadcasted_iota(jnp.int32, (_P, _P), 0)
        u = (q // _W).astype(jnp.float32)
        v = (q % _W).astype(jnp.float32)
        d0 = u - pos0
        d1 = v - pos1
        inside = (jnp.abs(u - r0) < 2.5) & (jnp.abs(v - r1) < 2.5)
        a_ref[:, :] = jnp.where(
            inside, sw * jnp.exp(-0.5 * (d0 * d0 + d1 * d1)), 0.0
        ).astype(jnp.bfloat16)

    out_ref[:, :] = jnp.dot(
        x_ref[:, :].astype(jnp.bfloat16), a_ref[:, :],
        preferred_element_type=jnp.float32,
        precision=jax.lax.Precision.DEFAULT,
    )


def kernel(input, pos2d, weight):
    b = input.shape[0]
    x = input.reshape(b, _P)
    params = jnp.stack(
        [pos2d[:, :, 0].reshape(_P), pos2d[:, :, 1].reshape(_P),
         weight.reshape(_P)], axis=0
    )
    params = jnp.pad(params, ((0, 5), (0, 0)))

    out = pl.pallas_call(
        _axs_kernel,
        grid=(b // _B_BLK,),
        in_specs=[
            pl.BlockSpec((8, _P), lambda i: (0, 0)),
            pl.BlockSpec((_B_BLK, _P), lambda i: (i, 0)),
        ],
        out_specs=pl.BlockSpec((_B_BLK, _P), lambda i: (i, 0)),
        out_shape=jax.ShapeDtypeStruct((b, _P), jnp.float32),
        scratch_shapes=[pltpu.VMEM((_P, _P), jnp.bfloat16)],
    )(params, x)
    return out.reshape(input.shape)


# speedup vs baseline: 1.0324x; 1.0324x over previous
import jax
import jax.numpy as jnp
from jax.experimental import pallas as pl
from jax.experimental.pallas import tpu as pltpu

_H = 28
_W = 28
_P = _H * _W


def _axs_kernel(params_ref, x_ref, out_ref, a_ref):
    pos0 = params_ref[0:1, :]
    pos1 = params_ref[1:2, :]
    sw = jnp.maximum(params_ref[2:3, :], 0.0)
    r0 = jnp.round(pos0)
    r1 = jnp.round(pos1)
    q = jax.lax.broadcasted_iota(jnp.int32, (_P, _P), 0)
    u = (q // _W).astype(jnp.float32)
    v = (q % _W).astype(jnp.float32)
    d0 = u - pos0
    d1 = v - pos1
    inside = (jnp.abs(u - r0) < 2.5) & (jnp.abs(v - r1) < 2.5)
    a_ref[:, :] = jnp.where(
        inside, sw * jnp.exp(-0.5 * (d0 * d0 + d1 * d1)), 0.0
    )

    out_ref[:, :] = jnp.dot(
        x_ref[:, :], a_ref[:, :],
        preferred_element_type=jnp.float32,
        precision=jax.lax.Precision.DEFAULT,
    )


def kernel(input, pos2d, weight):
    b = input.shape[0]
    x = input.reshape(b, _P)
    params = jnp.stack(
        [pos2d[:, :, 0].reshape(_P), pos2d[:, :, 1].reshape(_P),
         weight.reshape(_P)], axis=0
    )
    params = jnp.pad(params, ((0, 5), (0, 0)))

    out = pl.pallas_call(
        _axs_kernel,
        out_shape=jax.ShapeDtypeStruct((b, _P), jnp.float32),
        scratch_shapes=[pltpu.VMEM((_P, _P), jnp.float32)],
    )(params, x)
    return out.reshape(input.shape)


# speedup vs baseline: 1.0479x; 1.0150x over previous
import jax
import jax.numpy as jnp
from jax.experimental import pallas as pl
from jax.experimental.pallas import tpu as pltpu

_H = 28
_W = 28
_P = _H * _W


def _axs_kernel(pos_ref, w_ref, x_ref, out_ref, a_ref):
    def _flatten_rows(m):
        return jnp.concatenate([m[i:i + 1, :] for i in range(_H)], axis=1)

    pos0 = _flatten_rows(pos_ref[:, :, 0])
    pos1 = _flatten_rows(pos_ref[:, :, 1])
    sw = jnp.maximum(_flatten_rows(w_ref[:, :]), 0.0)
    r0 = jnp.round(pos0)
    r1 = jnp.round(pos1)
    q = jax.lax.broadcasted_iota(jnp.int32, (_P, _P), 0)
    u = (q // _W).astype(jnp.float32)
    v = (q % _W).astype(jnp.float32)
    d0 = u - pos0
    d1 = v - pos1
    inside = (jnp.abs(u - r0) < 2.5) & (jnp.abs(v - r1) < 2.5)
    a_ref[:, :] = jnp.where(
        inside, sw * jnp.exp(-0.5 * (d0 * d0 + d1 * d1)), 0.0
    )

    out_ref[:, :] = jnp.dot(
        x_ref[:, :], a_ref[:, :],
        preferred_element_type=jnp.float32,
        precision=jax.lax.Precision.DEFAULT,
    )


def kernel(input, pos2d, weight):
    b = input.shape[0]
    x = input.reshape(b, _P)

    out = pl.pallas_call(
        _axs_kernel,
        out_shape=jax.ShapeDtypeStruct((b, _P), jnp.float32),
        scratch_shapes=[pltpu.VMEM((_P, _P), jnp.float32)],
    )(pos2d, weight, x)
    return out.reshape(input.shape)
